# baseline (speedup 1.0000x reference)
import functools

import jax
import jax.numpy as jnp
from jax import lax
from jax.experimental import pallas as pl
from jax.experimental.pallas import tpu as pltpu
from jax.experimental.pallas import tpu_sc as plsc

_B, _H, _W, _M = 4, 512, 512, 8
_RI = 128
_NJ = _H // _RI
_RG = 8
_RW = 64


def _tc_body(w1t_ref, b1_ref, w2t_ref, b2_ref, x_ref, out_ref):
    w1t = w1t_ref[...]
    b1 = b1_ref[...]
    w2t = w2t_ref[...]
    b2 = b2_ref[...]
    for g in range(_RI // _RG):
        hts = []
        for rr in range(_RG):
            xtr = x_ref[0, :, g * _RG + rr, :]
            ht = jax.lax.dot_general(
                w1t, xtr, (((1,), (0,)), ((), ())),
                preferred_element_type=jnp.float32)
            hts.append(jnp.maximum(ht + b1, 0.0))
        for rr in range(_RG):
            lt = jax.lax.dot_general(
                w2t, hts[rr], (((1,), (0,)), ((), ())),
                preferred_element_type=jnp.float32)
            lt = lt + b2
            out_ref[0, pl.ds(g * _RG + rr, 1), :] = (
                lt[1:2, :] > lt[0:1, :]).astype(jnp.float32)


def _tc_pred(rgbp, W1, b1, W2, b2):
    return pl.pallas_call(
        _tc_body,
        grid=(_B, _NJ),
        in_specs=[
            pl.BlockSpec((64, 3), lambda b_, jj: (0, 0)),
            pl.BlockSpec((64, 1), lambda b_, jj: (0, 0)),
            pl.BlockSpec((2, 64), lambda b_, jj: (0, 0)),
            pl.BlockSpec((2, 1), lambda b_, jj: (0, 0)),
            pl.BlockSpec((1, 3, _RI, _W), lambda b_, jj: (b_, 0, jj, 0)),
        ],
        out_specs=pl.BlockSpec((1, _RI, _W), lambda b_, jj: (b_, jj, 0)),
        out_shape=jax.ShapeDtypeStruct((_B, _H, _W), jnp.float32),
    )(W1.T, b1.reshape(64, 1), W2.T, b2.reshape(2, 1), rgbp)


def _sc_label(predf, boxv):
    mesh = plsc.VectorSubcoreMesh(core_axis_name="c", subcore_axis_name="s")

    @functools.partial(
        pl.kernel, mesh=mesh,
        out_type=jax.ShapeDtypeStruct((_B, _H, _W), jnp.float32),
        compiler_params=pltpu.CompilerParams(needs_layout_passes=False),
        scratch_types=[
            pltpu.VMEM((_RW, _W), jnp.float32),
            pltpu.VMEM((_RW, _W), jnp.float32),
            pltpu.VMEM((_M * 5,), jnp.int32),
        ],
    )
    def sclab(pred_hbm, box_hbm, out_hbm, pv, lv_, bxv):
        wid = lax.axis_index("s") * 2 + lax.axis_index("c")
        bidx = lax.shift_right_logical(wid, 3)
        r0 = (wid & 7) * _RW
        iota16 = lax.iota(jnp.int32, 16)

        pltpu.sync_copy(box_hbm.at[pl.ds(bidx * (_M * 5), _M * 5)], bxv)
        pltpu.sync_copy(pred_hbm.at[bidx, pl.ds(r0, _RW)], pv)

        ones16 = jnp.ones((16,), jnp.float32)

        def initrow(i, _):
            lv_[lax.shift_right_logical(i, 5), pl.ds((i & 31) * 16, 16)] = (
                ones16)
            return 0

        lax.fori_loop(0, _RW * (_W // 16), initrow, 0)

        def scal(off):
            vec = plsc.load_gather(bxv, [jnp.full((16,), off, jnp.int32)])
            return vec[0]

        for m in range(_M):
            x1 = scal(m * 5 + 0)
            y1 = scal(m * 5 + 1)
            x2 = scal(m * 5 + 2)
            y2 = scal(m * 5 + 3)
            lvf = scal(m * 5 + 4).astype(jnp.float32)
            rlo = jnp.maximum(x1 - r0, 0)
            rhi = jnp.minimum(x2 - r0, _RW - 1)
            c1 = lax.shift_right_logical(y1, 4)
            c2 = lax.shift_right_logical(y2, 4)

            def rowbody(r, _):
                def colbody(c, _c):
                    u16 = c * 16 + iota16
                    msk = ((u16 >= y1) & (u16 <= y2)
                           & (pv[r, pl.ds(c * 16, 16)] > 0.0))
                    cur = lv_[r, pl.ds(c * 16, 16)]
                    lv_[r, pl.ds(c * 16, 16)] = jnp.where(msk, lvf, cur)
                    return 0

                lax.fori_loop(c1, c2 + 1, colbody, 0)
                return 0

            lax.fori_loop(rlo, rhi + 1, rowbody, 0)

        pltpu.sync_copy(lv_, out_hbm.at[bidx, pl.ds(r0, _RW)])

    return sclab(predf, boxv)


def kernel(rgb, depth, intrinsic, box, W1, b1, W2, b2):
    del depth, intrinsic
    rgbp = jnp.transpose(rgb, (0, 3, 1, 2))
    boxv = box.astype(jnp.int32).reshape(-1)
    predf = _tc_pred(rgbp, W1, b1, W2, b2)
    return _sc_label(predf, boxv)
